# baseline (speedup 1.0000x reference)
import math

import jax
import jax.numpy as jnp
from jax.experimental import pallas as pl
from jax.experimental.pallas import tpu as pltpu

B, Q, P, NE = 4, 256, 16384, 64
NO_ELECTRON_WEIGHT = 0.1
HUBER_DELTA = 0.1

TP = 2048
NPT = P // TP

_HIGHEST = jax.lax.Precision.HIGHEST


def _softplus(x):
    return jnp.log1p(jnp.exp(x))


def _lane_reduce_128(v):
    width = v.shape[1]
    while width > 128:
        width //= 2
        v = v[:, :width] + v[:, width:2 * width]
    return v


def _loss_kernel(mi_ref, logits_ref, pos_ref, chol_ref, tpos_ref,
                 seg_ref, trut_ref, acc_ref, total_ref, selpj_ref):
    b = pl.program_id(0)
    pt = pl.program_id(1)

    @pl.when(jnp.logical_and(b == 0, pt == 0))
    def _init():
        acc_ref[...] = jnp.zeros_like(acc_ref)

    @pl.when(pt == 0)
    def _per_batch_setup():
        pi = mi_ref[0, 0:1, :].astype(jnp.int32)
        ti = mi_ref[0, 1:2, :].astype(jnp.int32)

        iq = jax.lax.broadcasted_iota(jnp.int32, (Q, NE), 0)
        selp = jnp.where(iq == pi, 1.0, 0.0)
        ij = jax.lax.broadcasted_iota(jnp.int32, (NE, NE), 0)
        selt = jnp.where(ij == ti, 1.0, 0.0)
        selpj_ref[...] = jax.lax.dot_general(
            selp, selt, (((1,), (1,)), ((), ())))

        xrow = jnp.concatenate(
            [logits_ref[0, 0:1, :], logits_ref[0, 1:2, :]], axis=1)
        label_any = selpj_ref[...]
        xg = jax.lax.dot_general(xrow, label_any, (((1,), (0,)), ((), ())),
                                 precision=_HIGHEST)
        cls = (NO_ELECTRON_WEIGHT * jnp.sum(_softplus(xrow), axis=1,
                                            keepdims=True)
               + jnp.sum((1.0 - NO_ELECTRON_WEIGHT) * _softplus(xg) - xg,
                         axis=1, keepdims=True))
        acc_ref[b, 3:4, 0:1] += cls

        pos_b = pos_ref[...]
        ppt = jax.lax.dot_general(pos_b, selpj_ref[...],
                                  (((1,), (0,)), ((), ())),
                                  precision=_HIGHEST)
        cha = chol_ref[0]
        chb = chol_ref[1]
        ga = jax.lax.dot_general(cha, selpj_ref[...], (((1,), (0,)), ((), ())),
                                 precision=_HIGHEST)
        gb = jax.lax.dot_general(chb, selpj_ref[...], (((1,), (0,)), ((), ())),
                                 precision=_HIGHEST)
        i256 = jax.lax.broadcasted_iota(jnp.int32, (B * NE, NE), 0)
        je = jax.lax.broadcasted_iota(jnp.int32, (B * NE, NE), 1)
        selb = jnp.where(i256 == je + b * NE, 1.0, 0.0)
        tpt = jax.lax.dot_general(tpos_ref[...], selb, (((1,), (0,)), ((), ())),
                                  precision=_HIGHEST)

        d = tpt - ppt
        l00 = ga[0:1, :]
        l10 = gb[0:1, :]
        l11 = gb[1:2, :]
        z0 = d[0:1, :] / l00
        z1 = (d[1:2, :] - l10 * z0) / l11
        maha = z0 * z0 + z1 * z1
        logdet = jnp.log(l00) + jnp.log(l11)
        nll = 0.5 * maha + logdet + math.log(2.0 * math.pi)
        nll = jnp.clip(nll, -1e7, 1e7)
        acc_ref[b, 4:5, 0:1] += jnp.sum(nll, axis=1, keepdims=True)

        a = jnp.abs(d)
        huber = jnp.where(a < HUBER_DELTA, 0.5 * d * d,
                          HUBER_DELTA * (a - 0.5 * HUBER_DELTA))
        acc_ref[b, 5:6, 0:1] += jnp.sum(
            jnp.sum(huber, axis=1, keepdims=True), axis=0, keepdims=True)

    seg = seg_ref[0]
    tt = trut_ref[0]
    xt = jax.lax.dot_general(selpj_ref[...], seg, (((0,), (1,)), ((), ())))
    ex = jnp.exp(xt)
    lg = jnp.log1p(ex)
    c = lg - xt * tt
    s = jnp.sum(ex, axis=0, keepdims=True)
    n = jnp.sum(ex * tt, axis=0, keepdims=True)
    numl = n / s
    bq = jnp.sum(c, axis=0, keepdims=True)
    tden = jnp.sum(tt, axis=0, keepdims=True)
    acc_ref[b, 0:1, :] += _lane_reduce_128(bq)
    acc_ref[b, 1:2, :] += _lane_reduce_128(numl)
    acc_ref[b, 2:3, :] += _lane_reduce_128(tden)

    @pl.when(jnp.logical_and(b == B - 1, pt == NPT - 1))
    def _finalize():
        bce_sum = jnp.zeros((1, 1), jnp.float32)
        cls_sum = jnp.zeros((1, 1), jnp.float32)
        nll_sum = jnp.zeros((1, 1), jnp.float32)
        hub_sum = jnp.zeros((1, 1), jnp.float32)
        dice_sum = jnp.zeros((1, 1), jnp.float32)
        for bb in range(B):
            bce_sum += jnp.sum(acc_ref[bb, 0:1, :], axis=1, keepdims=True)
            num = 2.0 * jnp.sum(acc_ref[bb, 1:2, :], axis=1, keepdims=True)
            den = float(P) + jnp.sum(acc_ref[bb, 2:3, :], axis=1,
                                     keepdims=True)
            dice_sum += 1.0 - (num + 1.0) / (den + 1.0)
            cls_sum += acc_ref[bb, 3:4, 0:1]
            nll_sum += acc_ref[bb, 4:5, 0:1]
            hub_sum += acc_ref[bb, 5:6, 0:1]
        total = (cls_sum / (B * Q)
                 + bce_sum / (B * P * NE)
                 + dice_sum / B
                 + nll_sum / (B * NE)
                 + hub_sum / (B * NE * 2))
        total_ref[...] = total


def kernel(pred_logits, pred_seg_logits, true_seg, pred_positions,
           pred_std_cholesky, true_positions, query_batch_offsets,
           electron_batch_offsets, matched_indices):
    logits3 = pred_logits.reshape(B, 2, 128)
    pos_t = jnp.transpose(pred_positions)
    chol_t = jnp.transpose(pred_std_cholesky, (1, 2, 0))
    tpos_t = jnp.transpose(true_positions)
    true_t = jnp.transpose(true_seg, (0, 2, 1))

    grid = (B, NPT)
    acc, total = pl.pallas_call(
        _loss_kernel,
        grid=grid,
        in_specs=[
            pl.BlockSpec((1, 2, NE), lambda b, pt: (b, 0, 0)),
            pl.BlockSpec((1, 2, 128), lambda b, pt: (b, 0, 0)),
            pl.BlockSpec((2, Q), lambda b, pt: (0, b)),
            pl.BlockSpec((2, 2, Q), lambda b, pt: (0, 0, b)),
            pl.BlockSpec((2, B * NE), lambda b, pt: (0, 0)),
            pl.BlockSpec((1, TP, Q), lambda b, pt: (b, pt, 0)),
            pl.BlockSpec((1, NE, TP), lambda b, pt: (b, 0, pt)),
        ],
        out_specs=[
            pl.BlockSpec((B, 8, 128), lambda b, pt: (0, 0, 0)),
            pl.BlockSpec((1, 1), lambda b, pt: (0, 0)),
        ],
        out_shape=[
            jax.ShapeDtypeStruct((B, 8, 128), jnp.float32),
            jax.ShapeDtypeStruct((1, 1), jnp.float32),
        ],
        scratch_shapes=[
            pltpu.VMEM((Q, NE), jnp.float32),
        ],
    )(matched_indices, logits3, pos_t, chol_t, tpos_t,
      pred_seg_logits, true_t)
    return total[0, 0]


# speedup vs baseline: 1.3960x; 1.3960x over previous
import math

import jax
import jax.numpy as jnp
from jax.experimental import pallas as pl
from jax.experimental.pallas import tpu as pltpu

B, Q, P, NE = 4, 256, 16384, 64
NO_ELECTRON_WEIGHT = 0.1
HUBER_DELTA = 0.1

TP = 8192
NPT = P // TP

_HIGHEST = jax.lax.Precision.HIGHEST


def _softplus(x):
    return jnp.log1p(jnp.exp(x))


def _lane_reduce_128(v):
    width = v.shape[1]
    while width > 128:
        width //= 2
        v = v[:, :width] + v[:, width:2 * width]
    return v


def _loss_kernel(mi_ref, logits_ref, pos_ref, chol_ref, tpos_ref,
                 seg_ref, trut_ref, acc_ref, total_ref, selpj_ref):
    b = pl.program_id(0)
    pt = pl.program_id(1)

    @pl.when(jnp.logical_and(b == 0, pt == 0))
    def _init():
        acc_ref[...] = jnp.zeros_like(acc_ref)

    @pl.when(pt == 0)
    def _per_batch_setup():
        pi = mi_ref[0, 0:1, :].astype(jnp.int32)
        ti = mi_ref[0, 1:2, :].astype(jnp.int32)

        iq = jax.lax.broadcasted_iota(jnp.int32, (Q, NE), 0)
        selp = jnp.where(iq == pi, 1.0, 0.0)
        ij = jax.lax.broadcasted_iota(jnp.int32, (NE, NE), 0)
        selt = jnp.where(ij == ti, 1.0, 0.0)
        selpj_ref[...] = jax.lax.dot_general(
            selp, selt, (((1,), (1,)), ((), ())))

        xrow = jnp.concatenate(
            [logits_ref[0, 0:1, :], logits_ref[0, 1:2, :]], axis=1)
        label_any = selpj_ref[...]
        xg = jax.lax.dot_general(xrow, label_any, (((1,), (0,)), ((), ())),
                                 precision=_HIGHEST)
        cls = (NO_ELECTRON_WEIGHT * jnp.sum(_softplus(xrow), axis=1,
                                            keepdims=True)
               + jnp.sum((1.0 - NO_ELECTRON_WEIGHT) * _softplus(xg) - xg,
                         axis=1, keepdims=True))
        acc_ref[b, 3:4, 0:1] += cls

        pos_b = pos_ref[...]
        ppt = jax.lax.dot_general(pos_b, selpj_ref[...],
                                  (((1,), (0,)), ((), ())),
                                  precision=_HIGHEST)
        cha = chol_ref[0]
        chb = chol_ref[1]
        ga = jax.lax.dot_general(cha, selpj_ref[...], (((1,), (0,)), ((), ())),
                                 precision=_HIGHEST)
        gb = jax.lax.dot_general(chb, selpj_ref[...], (((1,), (0,)), ((), ())),
                                 precision=_HIGHEST)
        i256 = jax.lax.broadcasted_iota(jnp.int32, (B * NE, NE), 0)
        je = jax.lax.broadcasted_iota(jnp.int32, (B * NE, NE), 1)
        selb = jnp.where(i256 == je + b * NE, 1.0, 0.0)
        tpt = jax.lax.dot_general(tpos_ref[...], selb, (((1,), (0,)), ((), ())),
                                  precision=_HIGHEST)

        d = tpt - ppt
        l00 = ga[0:1, :]
        l10 = gb[0:1, :]
        l11 = gb[1:2, :]
        z0 = d[0:1, :] / l00
        z1 = (d[1:2, :] - l10 * z0) / l11
        maha = z0 * z0 + z1 * z1
        logdet = jnp.log(l00) + jnp.log(l11)
        nll = 0.5 * maha + logdet + math.log(2.0 * math.pi)
        nll = jnp.clip(nll, -1e7, 1e7)
        acc_ref[b, 4:5, 0:1] += jnp.sum(nll, axis=1, keepdims=True)

        a = jnp.abs(d)
        huber = jnp.where(a < HUBER_DELTA, 0.5 * d * d,
                          HUBER_DELTA * (a - 0.5 * HUBER_DELTA))
        acc_ref[b, 5:6, 0:1] += jnp.sum(
            jnp.sum(huber, axis=1, keepdims=True), axis=0, keepdims=True)

    seg = seg_ref[0]
    tt = trut_ref[0]
    xt = jax.lax.dot_general(selpj_ref[...], seg, (((0,), (1,)), ((), ())))
    ex = jnp.exp(xt)
    lg = jnp.log1p(ex)
    mstack = jnp.concatenate(
        [ex, ex * tt, lg - xt * tt, tt], axis=0)
    rsel = jnp.where(
        jax.lax.broadcasted_iota(jnp.int32, (8, 4 * NE), 1) // NE
        == jax.lax.broadcasted_iota(jnp.int32, (8, 4 * NE), 0),
        1.0, 0.0)
    red = jax.lax.dot_general(rsel, mstack, (((1,), (0,)), ((), ())))
    numl = red[1:2, :] / red[0:1, :]
    acc_ref[b, 0:1, :] += _lane_reduce_128(red[2:3, :])
    acc_ref[b, 1:2, :] += _lane_reduce_128(numl)
    acc_ref[b, 2:3, :] += _lane_reduce_128(red[3:4, :])

    @pl.when(jnp.logical_and(b == B - 1, pt == NPT - 1))
    def _finalize():
        bce_sum = jnp.zeros((1, 1), jnp.float32)
        cls_sum = jnp.zeros((1, 1), jnp.float32)
        nll_sum = jnp.zeros((1, 1), jnp.float32)
        hub_sum = jnp.zeros((1, 1), jnp.float32)
        dice_sum = jnp.zeros((1, 1), jnp.float32)
        for bb in range(B):
            bce_sum += jnp.sum(acc_ref[bb, 0:1, :], axis=1, keepdims=True)
            num = 2.0 * jnp.sum(acc_ref[bb, 1:2, :], axis=1, keepdims=True)
            den = float(P) + jnp.sum(acc_ref[bb, 2:3, :], axis=1,
                                     keepdims=True)
            dice_sum += 1.0 - (num + 1.0) / (den + 1.0)
            cls_sum += acc_ref[bb, 3:4, 0:1]
            nll_sum += acc_ref[bb, 4:5, 0:1]
            hub_sum += acc_ref[bb, 5:6, 0:1]
        total = (cls_sum / (B * Q)
                 + bce_sum / (B * P * NE)
                 + dice_sum / B
                 + nll_sum / (B * NE)
                 + hub_sum / (B * NE * 2))
        total_ref[...] = total


def kernel(pred_logits, pred_seg_logits, true_seg, pred_positions,
           pred_std_cholesky, true_positions, query_batch_offsets,
           electron_batch_offsets, matched_indices):
    logits3 = pred_logits.reshape(B, 2, 128)
    pos_t = jnp.transpose(pred_positions)
    chol_t = jnp.transpose(pred_std_cholesky, (1, 2, 0))
    tpos_t = jnp.transpose(true_positions)
    true_t = jnp.transpose(true_seg, (0, 2, 1))

    grid = (B, NPT)
    acc, total = pl.pallas_call(
        _loss_kernel,
        grid=grid,
        in_specs=[
            pl.BlockSpec((1, 2, NE), lambda b, pt: (b, 0, 0)),
            pl.BlockSpec((1, 2, 128), lambda b, pt: (b, 0, 0)),
            pl.BlockSpec((2, Q), lambda b, pt: (0, b)),
            pl.BlockSpec((2, 2, Q), lambda b, pt: (0, 0, b)),
            pl.BlockSpec((2, B * NE), lambda b, pt: (0, 0)),
            pl.BlockSpec((1, TP, Q), lambda b, pt: (b, pt, 0)),
            pl.BlockSpec((1, NE, TP), lambda b, pt: (b, 0, pt)),
        ],
        out_specs=[
            pl.BlockSpec((B, 8, 128), lambda b, pt: (0, 0, 0)),
            pl.BlockSpec((1, 1), lambda b, pt: (0, 0)),
        ],
        out_shape=[
            jax.ShapeDtypeStruct((B, 8, 128), jnp.float32),
            jax.ShapeDtypeStruct((1, 1), jnp.float32),
        ],
        scratch_shapes=[
            pltpu.VMEM((Q, NE), jnp.float32),
        ],
    )(matched_indices, logits3, pos_t, chol_t, tpos_t,
      pred_seg_logits, true_t)
    return total[0, 0]


# speedup vs baseline: 1.4202x; 1.0173x over previous
import math

import jax
import jax.numpy as jnp
from jax.experimental import pallas as pl
from jax.experimental.pallas import tpu as pltpu

B, Q, P, NE = 4, 256, 16384, 64
NO_ELECTRON_WEIGHT = 0.1
HUBER_DELTA = 0.1

TP = 16384
NPT = P // TP

_HIGHEST = jax.lax.Precision.HIGHEST


def _softplus(x):
    return jnp.log1p(jnp.exp(x))


def _lane_reduce_128(v):
    width = v.shape[1]
    while width > 128:
        width //= 2
        v = v[:, :width] + v[:, width:2 * width]
    return v


def _loss_kernel(mi_ref, logits_ref, pos_ref, chol_ref, tpos_ref,
                 seg_ref, trut_ref, acc_ref, total_ref, selpj_ref):
    b = pl.program_id(0)
    pt = pl.program_id(1)

    @pl.when(jnp.logical_and(b == 0, pt == 0))
    def _init():
        acc_ref[...] = jnp.zeros_like(acc_ref)

    @pl.when(pt == 0)
    def _per_batch_setup():
        pi = mi_ref[0, 0:1, :].astype(jnp.int32)
        ti = mi_ref[0, 1:2, :].astype(jnp.int32)

        iq = jax.lax.broadcasted_iota(jnp.int32, (Q, NE), 0)
        selp = jnp.where(iq == pi, 1.0, 0.0)
        ij = jax.lax.broadcasted_iota(jnp.int32, (NE, NE), 0)
        selt = jnp.where(ij == ti, 1.0, 0.0)
        selpj_ref[...] = jax.lax.dot_general(
            selp, selt, (((1,), (1,)), ((), ())))

        xrow = jnp.concatenate(
            [logits_ref[0, 0:1, :], logits_ref[0, 1:2, :]], axis=1)
        label_any = selpj_ref[...]
        xg = jax.lax.dot_general(xrow, label_any, (((1,), (0,)), ((), ())),
                                 precision=_HIGHEST)
        cls = (NO_ELECTRON_WEIGHT * jnp.sum(_softplus(xrow), axis=1,
                                            keepdims=True)
               + jnp.sum((1.0 - NO_ELECTRON_WEIGHT) * _softplus(xg) - xg,
                         axis=1, keepdims=True))
        acc_ref[b, 3:4, 0:1] += cls

        pos_b = pos_ref[...]
        ppt = jax.lax.dot_general(pos_b, selpj_ref[...],
                                  (((1,), (0,)), ((), ())),
                                  precision=_HIGHEST)
        cha = chol_ref[0]
        chb = chol_ref[1]
        ga = jax.lax.dot_general(cha, selpj_ref[...], (((1,), (0,)), ((), ())),
                                 precision=_HIGHEST)
        gb = jax.lax.dot_general(chb, selpj_ref[...], (((1,), (0,)), ((), ())),
                                 precision=_HIGHEST)
        i256 = jax.lax.broadcasted_iota(jnp.int32, (B * NE, NE), 0)
        je = jax.lax.broadcasted_iota(jnp.int32, (B * NE, NE), 1)
        selb = jnp.where(i256 == je + b * NE, 1.0, 0.0)
        tpt = jax.lax.dot_general(tpos_ref[...], selb, (((1,), (0,)), ((), ())),
                                  precision=_HIGHEST)

        d = tpt - ppt
        l00 = ga[0:1, :]
        l10 = gb[0:1, :]
        l11 = gb[1:2, :]
        z0 = d[0:1, :] / l00
        z1 = (d[1:2, :] - l10 * z0) / l11
        maha = z0 * z0 + z1 * z1
        logdet = jnp.log(l00) + jnp.log(l11)
        nll = 0.5 * maha + logdet + math.log(2.0 * math.pi)
        nll = jnp.clip(nll, -1e7, 1e7)
        acc_ref[b, 4:5, 0:1] += jnp.sum(nll, axis=1, keepdims=True)

        a = jnp.abs(d)
        huber = jnp.where(a < HUBER_DELTA, 0.5 * d * d,
                          HUBER_DELTA * (a - 0.5 * HUBER_DELTA))
        acc_ref[b, 5:6, 0:1] += jnp.sum(
            jnp.sum(huber, axis=1, keepdims=True), axis=0, keepdims=True)

    seg = seg_ref[0]
    tt = trut_ref[0]
    xt = jax.lax.dot_general(selpj_ref[...], seg, (((0,), (1,)), ((), ())))
    ex = jnp.exp(xt)
    lg = jnp.log1p(ex)
    mstack = jnp.concatenate(
        [ex, ex * tt, lg - xt * tt, tt], axis=0)
    rsel = jnp.where(
        jax.lax.broadcasted_iota(jnp.int32, (8, 4 * NE), 1) // NE
        == jax.lax.broadcasted_iota(jnp.int32, (8, 4 * NE), 0),
        1.0, 0.0)
    red = jax.lax.dot_general(rsel, mstack, (((1,), (0,)), ((), ())))
    numl = red[1:2, :] / red[0:1, :]
    acc_ref[b, 0:1, :] += _lane_reduce_128(red[2:3, :])
    acc_ref[b, 1:2, :] += _lane_reduce_128(numl)
    acc_ref[b, 2:3, :] += _lane_reduce_128(red[3:4, :])

    @pl.when(jnp.logical_and(b == B - 1, pt == NPT - 1))
    def _finalize():
        bce_sum = jnp.zeros((1, 1), jnp.float32)
        cls_sum = jnp.zeros((1, 1), jnp.float32)
        nll_sum = jnp.zeros((1, 1), jnp.float32)
        hub_sum = jnp.zeros((1, 1), jnp.float32)
        dice_sum = jnp.zeros((1, 1), jnp.float32)
        for bb in range(B):
            bce_sum += jnp.sum(acc_ref[bb, 0:1, :], axis=1, keepdims=True)
            num = 2.0 * jnp.sum(acc_ref[bb, 1:2, :], axis=1, keepdims=True)
            den = float(P) + jnp.sum(acc_ref[bb, 2:3, :], axis=1,
                                     keepdims=True)
            dice_sum += 1.0 - (num + 1.0) / (den + 1.0)
            cls_sum += acc_ref[bb, 3:4, 0:1]
            nll_sum += acc_ref[bb, 4:5, 0:1]
            hub_sum += acc_ref[bb, 5:6, 0:1]
        total = (cls_sum / (B * Q)
                 + bce_sum / (B * P * NE)
                 + dice_sum / B
                 + nll_sum / (B * NE)
                 + hub_sum / (B * NE * 2))
        total_ref[...] = total


def kernel(pred_logits, pred_seg_logits, true_seg, pred_positions,
           pred_std_cholesky, true_positions, query_batch_offsets,
           electron_batch_offsets, matched_indices):
    logits3 = pred_logits.reshape(B, 2, 128)
    pos_t = jnp.transpose(pred_positions)
    chol_t = jnp.transpose(pred_std_cholesky, (1, 2, 0))
    tpos_t = jnp.transpose(true_positions)
    true_t = jnp.transpose(true_seg, (0, 2, 1))

    grid = (B, NPT)
    acc, total = pl.pallas_call(
        _loss_kernel,
        grid=grid,
        in_specs=[
            pl.BlockSpec((1, 2, NE), lambda b, pt: (b, 0, 0)),
            pl.BlockSpec((1, 2, 128), lambda b, pt: (b, 0, 0)),
            pl.BlockSpec((2, Q), lambda b, pt: (0, b)),
            pl.BlockSpec((2, 2, Q), lambda b, pt: (0, 0, b)),
            pl.BlockSpec((2, B * NE), lambda b, pt: (0, 0)),
            pl.BlockSpec((1, TP, Q), lambda b, pt: (b, pt, 0)),
            pl.BlockSpec((1, NE, TP), lambda b, pt: (b, 0, pt)),
        ],
        out_specs=[
            pl.BlockSpec((B, 8, 128), lambda b, pt: (0, 0, 0)),
            pl.BlockSpec((1, 1), lambda b, pt: (0, 0)),
        ],
        out_shape=[
            jax.ShapeDtypeStruct((B, 8, 128), jnp.float32),
            jax.ShapeDtypeStruct((1, 1), jnp.float32),
        ],
        scratch_shapes=[
            pltpu.VMEM((Q, NE), jnp.float32),
        ],
    )(matched_indices, logits3, pos_t, chol_t, tpos_t,
      pred_seg_logits, true_t)
    return total[0, 0]
